# baseline (speedup 1.0000x reference)
import functools

import jax
import jax.numpy as jnp
from jax import lax
from jax.experimental import pallas as pl
from jax.experimental.pallas import tpu as pltpu
from jax.experimental.pallas import tpu_sc as plsc

C = 8
E = 128
L = 16
NC = 2
NS = 16
NW = NC * NS
NT = 16
ROWS = NT * C
NBUF = 4


def _mix_embed(x_hbm, qt_hbm, ch_hbm, out_hbm,
               xbuf, i0, i1, i2, i3, r0, r1, r2, r3, chbuf,
               g0, g1, g2, g3, w0, w1, w2, w3, *, B, T):
    TH = T // 2
    NSB = TH // NT
    idxb = [i0, i1, i2, i3]
    rowsb = [r0, r1, r2, r3]
    gsem = [g0, g1, g2, g3]
    wsem = [w0, w1, w2, w3]

    cid = lax.axis_index("c")
    sid = lax.axis_index("s")
    wid = sid * NC + cid
    b = wid // 2
    th = wid % 2
    t_base = th * TH

    for c in range(C):
        pltpu.sync_copy(x_hbm.at[pl.ds((b * C + c) * T + t_base, TH)],
                        xbuf.at[pl.ds(c * TH, TH)])
    pltpu.sync_copy(ch_hbm, chbuf)

    def make_idx(s, p):
        for k in range(ROWS // L):
            j = lax.iota(jnp.int32, L) + (k * L)
            pos = ((j & (C - 1)) * TH) + (j >> 3) + s * NT
            idxb[p][pl.ds(k * L, L)] = plsc.load_gather(xbuf, [pos])

    H = ROWS // 2

    def gather_start(p):
        pltpu.async_copy(qt_hbm.at[idxb[p].at[pl.ds(0, H)]],
                         rowsb[p].at[pl.ds(0, H)], gsem[p])
        pltpu.async_copy(qt_hbm.at[idxb[p].at[pl.ds(H, H)]],
                         rowsb[p].at[pl.ds(H, H)], gsem[p])

    def gather_wait(p):
        for o in (0, H):
            pltpu.make_async_copy(qt_hbm.at[idxb[p].at[pl.ds(o, H)]],
                                  rowsb[p].at[pl.ds(o, H)], gsem[p]).wait()

    def write_start(s, p):
        t0 = t_base + s * NT
        pltpu.async_copy(rowsb[p].reshape(NT, C * E),
                         out_hbm.at[b, pl.ds(t0, NT), :], wsem[p])

    def write_wait(p):
        pltpu.make_async_copy(rowsb[p].reshape(NT, C * E),
                              out_hbm.at[0, pl.ds(0, NT), :], wsem[p]).wait()

    def add_channel(p):
        ref = rowsb[p]
        for k2 in range(E // L):
            sl = pl.ds(k2 * L, L)
            chv = [chbuf[c, sl] for c in range(C)]

            def rowloop(rg, carry, ref=ref, sl=sl, chv=chv):
                rbase = rg * 16
                for rr in range(16):
                    ref[rbase + rr, sl] = (
                        ref[rbase + rr, sl] + chv[rr & (C - 1)])
                return carry

            lax.fori_loop(0, ROWS // 16, rowloop, 0)

    for p0 in range(2):
        make_idx(p0, p0)
        gather_start(p0)

    def outer(g, carry):
        for j in range(NBUF):
            s = g * NBUF + j
            pn = (j + 2) % NBUF

            @pl.when(s + 2 < NSB)
            def _prep(s=s, pn=pn):
                make_idx(s + 2, pn)

                @pl.when(s >= 2)
                def _drain(pn=pn):
                    write_wait(pn)

                gather_start(pn)

            gather_wait(j)
            add_channel(j)
            write_start(s, j)
        return carry

    lax.fori_loop(0, NSB // NBUF, outer, 0)
    for j in range(NBUF):
        write_wait(j)


def kernel(x, quant_table, channel_table):
    BC, T = x.shape
    B = BC // C
    mesh = plsc.VectorSubcoreMesh(core_axis_name="c", subcore_axis_name="s")
    k = functools.partial(
        pl.kernel,
        mesh=mesh,
        compiler_params=pltpu.CompilerParams(needs_layout_passes=False),
        out_type=jax.ShapeDtypeStruct((B, T, C * E), jnp.float32),
        scratch_types=[
            pltpu.VMEM((C * (T // 2),), jnp.int32),
            pltpu.VMEM((ROWS,), jnp.int32),
            pltpu.VMEM((ROWS,), jnp.int32),
            pltpu.VMEM((ROWS,), jnp.int32),
            pltpu.VMEM((ROWS,), jnp.int32),
            pltpu.VMEM((ROWS, E), jnp.float32),
            pltpu.VMEM((ROWS, E), jnp.float32),
            pltpu.VMEM((ROWS, E), jnp.float32),
            pltpu.VMEM((ROWS, E), jnp.float32),
            pltpu.VMEM((C, E), jnp.float32),
            pltpu.SemaphoreType.DMA,
            pltpu.SemaphoreType.DMA,
            pltpu.SemaphoreType.DMA,
            pltpu.SemaphoreType.DMA,
            pltpu.SemaphoreType.DMA,
            pltpu.SemaphoreType.DMA,
            pltpu.SemaphoreType.DMA,
            pltpu.SemaphoreType.DMA,
        ],
    )(functools.partial(_mix_embed, B=B, T=T))
    return k(x.reshape(-1), quant_table, channel_table)


# speedup vs baseline: 1.0340x; 1.0340x over previous
import functools

import jax
import jax.numpy as jnp
from jax import lax
from jax.experimental import pallas as pl
from jax.experimental.pallas import tpu as pltpu
from jax.experimental.pallas import tpu_sc as plsc

C = 8
E = 128
L = 16
NC = 2
NS = 16
NW = NC * NS
NT = 16
ROWS = NT * C
NBUF = 4


def _mix_embed(x_hbm, qt_hbm, ch_hbm, out_hbm,
               xbuf, i0, i1, i2, i3, r0, r1, r2, r3, chbuf,
               g0, g1, g2, g3, w0, w1, w2, w3, *, B, T):
    TH = T // 2
    NSB = TH // NT
    idxb = [i0, i1, i2, i3]
    rowsb = [r0, r1, r2, r3]
    gsem = [g0, g1, g2, g3]
    wsem = [w0, w1, w2, w3]

    cid = lax.axis_index("c")
    sid = lax.axis_index("s")
    wid = sid * NC + cid
    b = wid // 2
    th = wid % 2
    t_base = th * TH

    pltpu.sync_copy(x_hbm.at[pl.ds(b * C, C), pl.ds(t_base, TH)], xbuf)
    pltpu.sync_copy(ch_hbm, chbuf)

    def make_idx(s, p):
        for k in range(ROWS // L):
            j = lax.iota(jnp.int32, L) + (k * L)
            cvec = j & (C - 1)
            tvec = (j >> 3) + s * NT
            idxb[p][pl.ds(k * L, L)] = plsc.load_gather(xbuf, [cvec, tvec])

    def gather_start(p):
        pltpu.async_copy(qt_hbm.at[idxb[p]], rowsb[p], gsem[p])

    def gather_wait(p):
        pltpu.make_async_copy(qt_hbm.at[idxb[p]], rowsb[p], gsem[p]).wait()

    def write_start(s, p):
        t0 = t_base + s * NT
        pltpu.async_copy(rowsb[p].reshape(NT, C * E),
                         out_hbm.at[b, pl.ds(t0, NT), :], wsem[p])

    def write_wait(p):
        pltpu.make_async_copy(rowsb[p].reshape(NT, C * E),
                              out_hbm.at[0, pl.ds(0, NT), :], wsem[p]).wait()

    def add_channel(p):
        ref = rowsb[p]
        for k2 in range(E // L):
            sl = pl.ds(k2 * L, L)
            chv = [chbuf[c, sl] for c in range(C)]

            def rowloop(rg, carry, ref=ref, sl=sl, chv=chv):
                rbase = rg * 16
                for rr in range(16):
                    ref[rbase + rr, sl] = (
                        ref[rbase + rr, sl] + chv[rr & (C - 1)])
                return carry

            lax.fori_loop(0, ROWS // 16, rowloop, 0)

    for p0 in range(2):
        make_idx(p0, p0)
        gather_start(p0)

    def outer(g, carry):
        for j in range(NBUF):
            s = g * NBUF + j
            pn = (j + 2) % NBUF

            @pl.when(s + 2 < NSB)
            def _prep(s=s, pn=pn):
                make_idx(s + 2, pn)

                @pl.when(s >= 2)
                def _drain(pn=pn):
                    write_wait(pn)

                gather_start(pn)

            gather_wait(j)
            add_channel(j)
            write_start(s, j)
        return carry

    lax.fori_loop(0, NSB // NBUF, outer, 0)
    for j in range(NBUF):
        write_wait(j)


def kernel(x, quant_table, channel_table):
    BC, T = x.shape
    B = BC // C
    mesh = plsc.VectorSubcoreMesh(core_axis_name="c", subcore_axis_name="s")
    k = functools.partial(
        pl.kernel,
        mesh=mesh,
        compiler_params=pltpu.CompilerParams(needs_layout_passes=False),
        out_type=jax.ShapeDtypeStruct((B, T, C * E), jnp.float32),
        scratch_types=[
            pltpu.VMEM((C, T // 2), jnp.int32),
            pltpu.VMEM((ROWS,), jnp.int32),
            pltpu.VMEM((ROWS,), jnp.int32),
            pltpu.VMEM((ROWS,), jnp.int32),
            pltpu.VMEM((ROWS,), jnp.int32),
            pltpu.VMEM((ROWS, E), jnp.float32),
            pltpu.VMEM((ROWS, E), jnp.float32),
            pltpu.VMEM((ROWS, E), jnp.float32),
            pltpu.VMEM((ROWS, E), jnp.float32),
            pltpu.VMEM((C, E), jnp.float32),
            pltpu.SemaphoreType.DMA,
            pltpu.SemaphoreType.DMA,
            pltpu.SemaphoreType.DMA,
            pltpu.SemaphoreType.DMA,
            pltpu.SemaphoreType.DMA,
            pltpu.SemaphoreType.DMA,
            pltpu.SemaphoreType.DMA,
            pltpu.SemaphoreType.DMA,
        ],
    )(functools.partial(_mix_embed, B=B, T=T))
    return k(x, quant_table, channel_table)
